# baseline (speedup 1.0000x reference)
import functools

import jax
import jax.numpy as jnp
from jax import lax
from jax.experimental import pallas as pl
from jax.experimental.pallas import tpu as pltpu
from jax.experimental.pallas import tpu_sc as plsc

EPS = 1e-6
TOK_BLK = 256
TOPK = 2
BLK = 128


def _gelu_exact(y):
    return 0.5 * y * (1.0 + jax.lax.erf(y * 0.7071067811865476))


def _router_body(x_ref, wr_ref, c_ref, xn_ref, xnb_ref, aff_ref, sc_ref,
                 ei_ref):
    xb = x_ref[...]
    ms = jnp.mean(xb * xb, axis=-1, keepdims=True)
    xn = wr_ref[...] * (xb * jax.lax.rsqrt(ms + EPS))
    xn_ref[...] = xn
    xnb_ref[...] = xn.astype(jnp.bfloat16)
    logits = jax.lax.dot_general(
        xn, c_ref[...], (((1,), (1,)), ((), ())),
        preferred_element_type=jnp.float32)
    m = jnp.max(logits, axis=-1, keepdims=True)
    ex = jnp.exp(logits - m)
    aff = ex / jnp.sum(ex, axis=-1, keepdims=True)
    aff_ref[...] = aff
    ne = aff.shape[-1]
    idx = jax.lax.broadcasted_iota(jnp.int32, aff.shape, 1)
    m1 = jnp.max(aff, axis=-1, keepdims=True)
    i1 = jnp.min(jnp.where(aff == m1, idx, ne), axis=-1, keepdims=True)
    oh1 = idx == i1
    a2 = jnp.where(oh1, -jnp.inf, aff)
    m2 = jnp.max(a2, axis=-1, keepdims=True)
    i2 = jnp.min(jnp.where(a2 == m2, idx, ne), axis=-1, keepdims=True)
    sc_ref[...] = jnp.concatenate([m1, m2], axis=1)
    ei_ref[...] = jnp.concatenate([i1, i2], axis=1)


def _shared_body(xnb_ref, w1_ref, w2_ref, bs1_ref, bs2_ref, g_ref):
    x = xnb_ref[...]
    h = jax.lax.dot_general(
        x, w1_ref[0], (((1,), (0,)), ((), ())),
        preferred_element_type=jnp.float32)
    h = h + bs1_ref[...]
    y = jax.lax.dot_general(
        h.astype(jnp.bfloat16), w2_ref[0], (((1,), (0,)), ((), ())),
        preferred_element_type=jnp.float32)
    y = y + bs2_ref[...]
    g_ref[...] = _gelu_exact(y)


def _make_s1(s, ne, p, padq, nb, nspec, tok_off=0, interpret=False):
    n_tiles = 16
    per = p // n_tiles
    nvec = per // 16
    zper = padq // n_tiles
    nsc = per // 128
    mesh = plsc.VectorSubcoreMesh(
        core_axis_name="c", subcore_axis_name="s", num_cores=1)

    @functools.partial(
        pl.kernel,
        out_type=[
            jax.ShapeDtypeStruct((p,), jnp.int32),
            jax.ShapeDtypeStruct((padq,), jnp.int32),
            jax.ShapeDtypeStruct((16,), jnp.int32),
        ],
        mesh=mesh,
        interpret=interpret,
        compiler_params=pltpu.CompilerParams(needs_layout_passes=False),
        scratch_types=[
            pltpu.VMEM((per,), jnp.int32),
            pltpu.VMEM((per,), jnp.int32),
            pltpu.VMEM((nsc, 128), jnp.int32),
            pltpu.VMEM((nsc, 128), jnp.int32),
            pltpu.VMEM((16,), jnp.int32),
            pltpu.VMEM((16,), jnp.int32),
            pltpu.VMEM((zper,), jnp.int32),
            pltpu.VMEM((16, 16), jnp.int32),
            pltpu.VMEM_SHARED((16, 16), jnp.int32),
            pltpu.VMEM_SHARED((padq,), jnp.int32),
        ],
    )
    def s1(ei_hbm, pos_hbm, rowtok_hbm, spec_hbm, e_vm, pos_vm, pos2_vm,
           tok2_vm, row_vm, base_vm, zero_vm, allcnt_vm, counts_sh,
           rowtok_sh):
        wid = lax.axis_index("s")
        base = wid * per
        pltpu.sync_copy(ei_hbm.at[pl.ds(base, per)], e_vm)
        iota = lax.iota(jnp.int32, 16)
        zv = jnp.zeros((16,), jnp.int32)
        ones = jnp.full((16,), 1, jnp.int32)
        c15 = jnp.full((16,), 15, jnp.int32)

        def cvc(x):
            return jnp.full((16,), x, jnp.int32)

        for w in range(n_tiles):
            @pl.when(wid == w)
            def _(w=w):
                row_vm[...] = cvc(w)
        wid_vec = row_vm[...]

        ridx = {k: (iota - cvc(k)) & c15 for k in range(1, 16)}
        geq = {k: iota >= cvc(k) for k in range(1, 16)}

        def rotg(k):
            return plsc.load_gather(row_vm, [ridx[k]])

        def allred(x):
            for k in (1, 2, 4, 8):
                row_vm[...] = x
                x = x + rotg(k)
            return x

        def prefix_incl(x):
            for k in (1, 2, 4, 8):
                row_vm[...] = x
                g = rotg(k)
                x = x + jnp.where(geq[k], g, zv)
            return x

        acc = [zv for _ in range(ne)]
        for i in range(nvec):
            v = e_vm[pl.ds(i * 16, 16)]
            for e in range(ne):
                acc[e] = acc[e] + jnp.where(v == cvc(e), ones, zv)
        hist = zv
        for e in range(ne):
            hist = hist + jnp.where(iota == cvc(e), allred(acc[e]), zv)
        row_vm[...] = hist
        pltpu.sync_copy(row_vm, counts_sh.at[wid])
        for i in range(zper // 16):
            zero_vm[pl.ds(i * 16, 16)] = zv
        plsc.subcore_barrier()
        pltpu.sync_copy(counts_sh, allcnt_vm)
        totals = zv
        prior = zv
        for w in range(n_tiles):
            vw = allcnt_vm[w]
            totals = totals + vw
            prior = prior + vw * jnp.where(cvc(w) < wid_vec, ones, zv)
        nblk = lax.shift_right_logical(totals + cvc(BLK - 1), cvc(7))
        cb_inc = prefix_incl(nblk)
        cb_exc = cb_inc - nblk
        base_vec = cb_exc * cvc(BLK) + prior
        base_vm[...] = cb_inc
        nact_v = plsc.load_gather(base_vm, [cvc(ne - 1)])
        base_vm[...] = base_vec
        for i in range(nvec):
            v = e_vm[pl.ds(i * 16, 16)]
            bv = plsc.load_gather(base_vm, [v])
            row_vm[...] = v
            rank = zv
            cntv = ones
            for k in range(1, 16):
                eq = jnp.where(rotg(k) == v, ones, zv)
                rank = rank + jnp.where(geq[k], eq, zv)
                cntv = cntv + eq
            posv = bv + rank
            plsc.store_scatter(base_vm, [v], bv + cntv)
            pos_vm[pl.ds(i * 16, 16)] = posv
            r, c0 = divmod(i * 16, 128)
            pos2_vm[r, pl.ds(c0, 16)] = posv
            tok2_vm[r, pl.ds(c0, 16)] = lax.shift_right_logical(
                wid_vec * cvc(per) + cvc(i * 16) + iota, ones) + cvc(tok_off)
        pltpu.sync_copy(pos_vm, pos_hbm.at[pl.ds(base, per)])
        pltpu.sync_copy(zero_vm, rowtok_sh.at[pl.ds(wid * zper, zper)])
        plsc.subcore_barrier()
        for ch in range(nsc):
            pltpu.sync_copy(tok2_vm.at[ch], rowtok_sh.at[pos2_vm.at[ch]])
        plsc.subcore_barrier()

        @pl.when(wid == 0)
        def _():
            pltpu.sync_copy(rowtok_sh, rowtok_hbm)
            zero_vm[pl.ds(0, 16)] = cb_inc
            pltpu.sync_copy(zero_vm.at[pl.ds(0, 16)], spec_hbm)

    return s1


def _make_gather(n_rows, d, rows_per, chunk, dtype, interpret=False):
    nchunk = rows_per // chunk
    mesh = plsc.VectorSubcoreMesh(core_axis_name="c", subcore_axis_name="s")

    @functools.partial(
        pl.kernel,
        out_type=jax.ShapeDtypeStruct((n_rows, d), dtype),
        mesh=mesh,
        interpret=interpret,
        compiler_params=pltpu.CompilerParams(needs_layout_passes=False),
        scratch_types=[
            pltpu.VMEM((rows_per,), jnp.int32),
            pltpu.VMEM((chunk, d), dtype),
            pltpu.VMEM((chunk, d), dtype),
            pltpu.SemaphoreType.DMA,
            pltpu.SemaphoreType.DMA,
        ],
    )
    def gk(table_hbm, idx_hbm, out_hbm, idx_vm, buf_a, buf_b, sem_a, sem_b):
        wid = lax.axis_index("s") * 2 + lax.axis_index("c")
        base = wid * rows_per
        pltpu.sync_copy(idx_hbm.at[pl.ds(base, rows_per)], idx_vm)
        bufs = (buf_a, buf_b)
        sems = (sem_a, sem_b)
        cps = []
        for ch in range(nchunk):
            cps.append(pltpu.async_copy(
                table_hbm.at[idx_vm.at[pl.ds(ch * chunk, chunk)]],
                bufs[ch % 2], sems[ch % 2]))
            if ch >= 1:
                cps[ch - 1].wait()
                pltpu.sync_copy(
                    bufs[(ch - 1) % 2],
                    out_hbm.at[pl.ds(base + (ch - 1) * chunk, chunk)])
        cps[nchunk - 1].wait()
        pltpu.sync_copy(
            bufs[(nchunk - 1) % 2],
            out_hbm.at[pl.ds(base + (nchunk - 1) * chunk, chunk)])

    return gk


def _expert_of(j, spec_ref, ne):
    e = jnp.int32(0)
    for k in range(ne - 1):
        e = e + jnp.where(j >= spec_ref[k], 1, 0).astype(jnp.int32)
    return e


def _grouped_body(spec_ref, xg_ref, w1_ref, w2_ref, br1_ref, br2_ref, y_ref):
    j = pl.program_id(0)
    ne = br1_ref.shape[0]
    nact = spec_ref[ne - 1]
    be = _expert_of(j, spec_ref, ne)

    @pl.when(j < nact)
    def _():
        x = xg_ref[...].astype(jnp.bfloat16)
        h = jax.lax.dot_general(
            x, w1_ref[0], (((1,), (0,)), ((), ())),
            preferred_element_type=jnp.float32)
        h = h + br1_ref[pl.ds(be, 1), :]
        y = jax.lax.dot_general(
            h.astype(jnp.bfloat16), w2_ref[0], (((1,), (0,)), ((), ())),
            preferred_element_type=jnp.float32)
        y = y + br2_ref[pl.ds(be, 1), :]
        y_ref[...] = _gelu_exact(y)


def _combine_body(xn_ref, shg_ref, sc_ref, y2_ref, out_ref):
    d = xn_ref.shape[1]
    s0 = sc_ref[:, 0:1]
    s1 = sc_ref[:, 1:2]
    ya = y2_ref[:, :d]
    yb = y2_ref[:, d:]
    out_ref[...] = xn_ref[...] + shg_ref[...] + s0 * ya + s1 * yb


def _impl(x, w_rms, Ws1, bs1, Ws2, bs2, Wr1, br1, Wr2, br2, centroids,
          interpret):
    b, s, d = x.shape
    ne, _, e = Wr1.shape
    xm = x.reshape(s, d)
    nt = s // TOK_BLK
    p = s * TOPK
    nb = p // BLK + ne
    padq = nb * BLK
    nspec = ((nb + 1 + 15) // 16) * 16

    xn, xnb, aff, scores, eidx = pl.pallas_call(
        _router_body,
        grid=(nt,),
        in_specs=[
            pl.BlockSpec((TOK_BLK, d), lambda t: (t, 0)),
            pl.BlockSpec((1, d), lambda t: (0, 0)),
            pl.BlockSpec((ne, d), lambda t: (0, 0)),
        ],
        out_specs=[
            pl.BlockSpec((TOK_BLK, d), lambda t: (t, 0)),
            pl.BlockSpec((TOK_BLK, d), lambda t: (t, 0)),
            pl.BlockSpec((TOK_BLK, ne), lambda t: (t, 0)),
            pl.BlockSpec((TOK_BLK, TOPK), lambda t: (t, 0)),
            pl.BlockSpec((TOK_BLK, TOPK), lambda t: (t, 0)),
        ],
        out_shape=[
            jax.ShapeDtypeStruct((s, d), jnp.float32),
            jax.ShapeDtypeStruct((s, d), jnp.bfloat16),
            jax.ShapeDtypeStruct((s, ne), jnp.float32),
            jax.ShapeDtypeStruct((s, TOPK), jnp.float32),
            jax.ShapeDtypeStruct((s, TOPK), jnp.int32),
        ],
        interpret=interpret,
    )(xm, w_rms.reshape(1, d), centroids)

    w1b = Wr1.astype(jnp.bfloat16)
    w2b = Wr2.astype(jnp.bfloat16)
    ws1b = Ws1.astype(jnp.bfloat16)
    ws2b = Ws2.astype(jnp.bfloat16)

    shg = pl.pallas_call(
        _shared_body,
        grid=(nt,),
        in_specs=[
            pl.BlockSpec((TOK_BLK, d), lambda t: (t, 0)),
            pl.BlockSpec((1, d, e), lambda t: (0, 0, 0)),
            pl.BlockSpec((1, e, d), lambda t: (0, 0, 0)),
            pl.BlockSpec((1, e), lambda t: (0, 0)),
            pl.BlockSpec((1, d), lambda t: (0, 0)),
        ],
        out_specs=pl.BlockSpec((TOK_BLK, d), lambda t: (t, 0)),
        out_shape=jax.ShapeDtypeStruct((s, d), jnp.float32),
        interpret=interpret,
    )(xnb, ws1b, ws2b, bs1, bs2)

    nh = 2
    sh = s // nh
    ph = sh * TOPK
    nbh = ph // BLK + ne
    padqh = nbh * BLK
    y2s = []
    eidx2 = eidx.reshape(nh, ph)
    for h in range(nh):
        s1k = _make_s1(sh, ne, ph, padqh, nbh, nspec, sh * h, interpret)
        pos, rowtok, spec = s1k(eidx2[h])

        s2k = _make_gather(padqh, d, padqh // 32, padqh // 64, jnp.float32,
                           interpret)
        xg = s2k(xn, rowtok)

        y = pl.pallas_call(
            _grouped_body,
            grid_spec=pltpu.PrefetchScalarGridSpec(
                num_scalar_prefetch=1,
                grid=(nbh,),
                in_specs=[
                    pl.BlockSpec((BLK, d), lambda j, spec: (j, 0)),
                    pl.BlockSpec(
                        (1, d, e),
                        lambda j, spec: (_expert_of(j, spec, ne), 0, 0)),
                    pl.BlockSpec(
                        (1, e, d),
                        lambda j, spec: (_expert_of(j, spec, ne), 0, 0)),
                    pl.BlockSpec((ne, e), lambda j, spec: (0, 0)),
                    pl.BlockSpec((ne, d), lambda j, spec: (0, 0)),
                ],
                out_specs=pl.BlockSpec((BLK, d), lambda j, spec: (j, 0)),
            ),
            out_shape=jax.ShapeDtypeStruct((padqh, d), jnp.float32),
            interpret=interpret,
        )(spec, xg, w1b, w2b, br1, br2)

        s3k = _make_gather(ph, d, ph // 32, ph // 64, jnp.float32, interpret)
        y2s.append(s3k(y, pos))

    y2 = jnp.concatenate(y2s, axis=0)

    out = pl.pallas_call(
        _combine_body,
        grid=(nt,),
        in_specs=[
            pl.BlockSpec((TOK_BLK, d), lambda t: (t, 0)),
            pl.BlockSpec((TOK_BLK, d), lambda t: (t, 0)),
            pl.BlockSpec((TOK_BLK, TOPK), lambda t: (t, 0)),
            pl.BlockSpec((TOK_BLK, TOPK * d), lambda t: (t, 0)),
        ],
        out_specs=pl.BlockSpec((TOK_BLK, d), lambda t: (t, 0)),
        out_shape=jax.ShapeDtypeStruct((s, d), jnp.float32),
        interpret=interpret,
    )(xn, shg, scores, y2.reshape(s, TOPK * d))

    return out.reshape(b, s, d), aff


def kernel(x, w_rms, Ws1, bs1, Ws2, bs2, Wr1, br1, Wr2, br2, centroids):
    return _impl(x, w_rms, Ws1, bs1, Ws2, bs2, Wr1, br1, Wr2, br2, centroids,
                 interpret=False)


# speedup vs baseline: 1.5455x; 1.5455x over previous
import functools

import jax
import jax.numpy as jnp
from jax import lax
from jax.experimental import pallas as pl
from jax.experimental.pallas import tpu as pltpu
from jax.experimental.pallas import tpu_sc as plsc

EPS = 1e-6
TOK_BLK = 256
TOPK = 2
BLK = 128


def _gelu_exact(y):
    return 0.5 * y * (1.0 + jax.lax.erf(y * 0.7071067811865476))


def _router_body(x_ref, wr_ref, c_ref, xn_ref, xnb_ref, aff_ref, sc_ref,
                 ei_ref):
    xb = x_ref[...]
    ms = jnp.mean(xb * xb, axis=-1, keepdims=True)
    xn = wr_ref[...] * (xb * jax.lax.rsqrt(ms + EPS))
    xn_ref[...] = xn
    xnb_ref[...] = xn.astype(jnp.bfloat16)
    logits = jax.lax.dot_general(
        xn, c_ref[...], (((1,), (1,)), ((), ())),
        preferred_element_type=jnp.float32)
    m = jnp.max(logits, axis=-1, keepdims=True)
    ex = jnp.exp(logits - m)
    aff = ex / jnp.sum(ex, axis=-1, keepdims=True)
    aff_ref[...] = aff
    ne = aff.shape[-1]
    idx = jax.lax.broadcasted_iota(jnp.int32, aff.shape, 1)
    m1 = jnp.max(aff, axis=-1, keepdims=True)
    i1 = jnp.min(jnp.where(aff == m1, idx, ne), axis=-1, keepdims=True)
    oh1 = idx == i1
    a2 = jnp.where(oh1, -jnp.inf, aff)
    m2 = jnp.max(a2, axis=-1, keepdims=True)
    i2 = jnp.min(jnp.where(a2 == m2, idx, ne), axis=-1, keepdims=True)
    sc_ref[...] = jnp.concatenate([m1, m2], axis=1)
    ei_ref[...] = jnp.concatenate([i1, i2], axis=1)


def _shared_body(xnb_ref, w1_ref, w2_ref, bs1_ref, bs2_ref, g_ref):
    x = xnb_ref[...]
    h = jax.lax.dot_general(
        x, w1_ref[0], (((1,), (0,)), ((), ())),
        preferred_element_type=jnp.float32)
    h = h + bs1_ref[...]
    y = jax.lax.dot_general(
        h.astype(jnp.bfloat16), w2_ref[0], (((1,), (0,)), ((), ())),
        preferred_element_type=jnp.float32)
    y = y + bs2_ref[...]
    g_ref[...] = _gelu_exact(y)


def _make_s1(s, ne, p, padq, nb, nspec, tok_off=0, interpret=False):
    n_tiles = 16
    per = p // n_tiles
    nvec = per // 16
    zper = padq // n_tiles
    nsc = per // 128
    mesh = plsc.VectorSubcoreMesh(
        core_axis_name="c", subcore_axis_name="s", num_cores=1)

    @functools.partial(
        pl.kernel,
        out_type=[
            jax.ShapeDtypeStruct((p,), jnp.int32),
            jax.ShapeDtypeStruct((padq,), jnp.int32),
            jax.ShapeDtypeStruct((16,), jnp.int32),
        ],
        mesh=mesh,
        interpret=interpret,
        compiler_params=pltpu.CompilerParams(needs_layout_passes=False),
        scratch_types=[
            pltpu.VMEM((per,), jnp.int32),
            pltpu.VMEM((per,), jnp.int32),
            pltpu.VMEM((nsc, 128), jnp.int32),
            pltpu.VMEM((nsc, 128), jnp.int32),
            pltpu.VMEM((16,), jnp.int32),
            pltpu.VMEM((16,), jnp.int32),
            pltpu.VMEM((zper,), jnp.int32),
            pltpu.VMEM((16, 16), jnp.int32),
            pltpu.VMEM_SHARED((16, 16), jnp.int32),
            pltpu.VMEM_SHARED((padq,), jnp.int32),
        ],
    )
    def s1(ei_hbm, pos_hbm, rowtok_hbm, spec_hbm, e_vm, pos_vm, pos2_vm,
           tok2_vm, row_vm, base_vm, zero_vm, allcnt_vm, counts_sh,
           rowtok_sh):
        wid = lax.axis_index("s")
        base = wid * per
        pltpu.sync_copy(ei_hbm.at[pl.ds(base, per)], e_vm)
        iota = lax.iota(jnp.int32, 16)
        zv = jnp.zeros((16,), jnp.int32)
        ones = jnp.full((16,), 1, jnp.int32)
        c15 = jnp.full((16,), 15, jnp.int32)

        def cvc(x):
            return jnp.full((16,), x, jnp.int32)

        for w in range(n_tiles):
            @pl.when(wid == w)
            def _(w=w):
                row_vm[...] = cvc(w)
        wid_vec = row_vm[...]

        ridx = {k: (iota - cvc(k)) & c15 for k in range(1, 16)}
        geq = {k: iota >= cvc(k) for k in range(1, 16)}

        def rotg(k):
            return plsc.load_gather(row_vm, [ridx[k]])

        def allred(x):
            for k in (1, 2, 4, 8):
                row_vm[...] = x
                x = x + rotg(k)
            return x

        def prefix_incl(x):
            for k in (1, 2, 4, 8):
                row_vm[...] = x
                g = rotg(k)
                x = x + jnp.where(geq[k], g, zv)
            return x

        acc = [zv for _ in range(ne)]
        for i in range(nvec):
            v = e_vm[pl.ds(i * 16, 16)]
            for e in range(ne):
                acc[e] = acc[e] + jnp.where(v == cvc(e), ones, zv)
        hist = zv
        for e in range(ne):
            hist = hist + jnp.where(iota == cvc(e), allred(acc[e]), zv)
        row_vm[...] = hist
        pltpu.sync_copy(row_vm, counts_sh.at[wid])
        for i in range(zper // 16):
            zero_vm[pl.ds(i * 16, 16)] = zv
        plsc.subcore_barrier()
        pltpu.sync_copy(counts_sh, allcnt_vm)
        totals = zv
        prior = zv
        for w in range(n_tiles):
            vw = allcnt_vm[w]
            totals = totals + vw
            prior = prior + vw * jnp.where(cvc(w) < wid_vec, ones, zv)
        nblk = lax.shift_right_logical(totals + cvc(BLK - 1), cvc(7))
        cb_inc = prefix_incl(nblk)
        cb_exc = cb_inc - nblk
        base_vec = cb_exc * cvc(BLK) + prior
        base_vm[...] = cb_inc
        nact_v = plsc.load_gather(base_vm, [cvc(ne - 1)])
        base_vm[...] = base_vec
        for i in range(nvec):
            v = e_vm[pl.ds(i * 16, 16)]
            bv = plsc.load_gather(base_vm, [v])
            row_vm[...] = v
            rank = zv
            cntv = ones
            for k in range(1, 16):
                eq = jnp.where(rotg(k) == v, ones, zv)
                rank = rank + jnp.where(geq[k], eq, zv)
                cntv = cntv + eq
            posv = bv + rank
            plsc.store_scatter(base_vm, [v], bv + cntv)
            pos_vm[pl.ds(i * 16, 16)] = posv
            r, c0 = divmod(i * 16, 128)
            pos2_vm[r, pl.ds(c0, 16)] = posv
            tok2_vm[r, pl.ds(c0, 16)] = lax.shift_right_logical(
                wid_vec * cvc(per) + cvc(i * 16) + iota, ones) + cvc(tok_off)
        pltpu.sync_copy(pos_vm, pos_hbm.at[pl.ds(base, per)])
        pltpu.sync_copy(zero_vm, rowtok_sh.at[pl.ds(wid * zper, zper)])
        plsc.subcore_barrier()
        for ch in range(nsc):
            pltpu.sync_copy(tok2_vm.at[ch], rowtok_sh.at[pos2_vm.at[ch]])
        plsc.subcore_barrier()

        @pl.when(wid == 0)
        def _():
            pltpu.sync_copy(rowtok_sh, rowtok_hbm)
            zero_vm[pl.ds(0, 16)] = cb_inc
            pltpu.sync_copy(zero_vm.at[pl.ds(0, 16)], spec_hbm)

    return s1


def _make_gather(n_rows, d, rows_per, chunk, dtype, interpret=False):
    nchunk = rows_per // chunk
    mesh = plsc.VectorSubcoreMesh(core_axis_name="c", subcore_axis_name="s")

    @functools.partial(
        pl.kernel,
        out_type=jax.ShapeDtypeStruct((n_rows, d), dtype),
        mesh=mesh,
        interpret=interpret,
        compiler_params=pltpu.CompilerParams(needs_layout_passes=False),
        scratch_types=[
            pltpu.VMEM((rows_per,), jnp.int32),
            pltpu.VMEM((chunk, d), dtype),
            pltpu.VMEM((chunk, d), dtype),
            pltpu.SemaphoreType.DMA,
            pltpu.SemaphoreType.DMA,
        ],
    )
    def gk(table_hbm, idx_hbm, out_hbm, idx_vm, buf_a, buf_b, sem_a, sem_b):
        wid = lax.axis_index("s") * 2 + lax.axis_index("c")
        base = wid * rows_per
        pltpu.sync_copy(idx_hbm.at[pl.ds(base, rows_per)], idx_vm)
        bufs = (buf_a, buf_b)
        sems = (sem_a, sem_b)
        cps = []
        for ch in range(nchunk):
            cps.append(pltpu.async_copy(
                table_hbm.at[idx_vm.at[pl.ds(ch * chunk, chunk)]],
                bufs[ch % 2], sems[ch % 2]))
            if ch >= 1:
                cps[ch - 1].wait()
                pltpu.sync_copy(
                    bufs[(ch - 1) % 2],
                    out_hbm.at[pl.ds(base + (ch - 1) * chunk, chunk)])
        cps[nchunk - 1].wait()
        pltpu.sync_copy(
            bufs[(nchunk - 1) % 2],
            out_hbm.at[pl.ds(base + (nchunk - 1) * chunk, chunk)])

    return gk


def _expert_of(j, spec_ref, ne):
    e = jnp.int32(0)
    for k in range(ne - 1):
        e = e + jnp.where(j >= spec_ref[k], 1, 0).astype(jnp.int32)
    return e


def _grouped_body(spec_ref, rt_ref, xnb_ref, w1_ref, w2_ref, br1_ref,
                  br2_ref, y_ref):
    j = pl.program_id(0)
    ne = br1_ref.shape[0]
    nact = spec_ref[ne - 1]
    be = _expert_of(j, spec_ref, ne)

    @pl.when(j < nact)
    def _():
        rt = rt_ref[...]
        s = xnb_ref.shape[0]
        ioty = jax.lax.broadcasted_iota(jnp.int32, (rt.shape[0], s), 1)
        oh = (ioty == rt).astype(jnp.bfloat16)
        xg = jax.lax.dot_general(
            oh, xnb_ref[...], (((1,), (0,)), ((), ())),
            preferred_element_type=jnp.float32).astype(jnp.bfloat16)
        h = jax.lax.dot_general(
            xg, w1_ref[0], (((1,), (0,)), ((), ())),
            preferred_element_type=jnp.float32)
        h = h + br1_ref[pl.ds(be, 1), :]
        y = jax.lax.dot_general(
            h.astype(jnp.bfloat16), w2_ref[0], (((1,), (0,)), ((), ())),
            preferred_element_type=jnp.float32)
        y = y + br2_ref[pl.ds(be, 1), :]
        y_ref[...] = _gelu_exact(y).astype(jnp.bfloat16)


def _combine_body(xn_ref, shg_ref, sc_ref, pos_ref, y_ref, out_ref):
    padq = y_ref.shape[0]
    blk = xn_ref.shape[0]
    s0 = sc_ref[:, 0:1]
    s1 = sc_ref[:, 1:2]
    pa = pos_ref[:, 0:1]
    pb = pos_ref[:, 1:2]
    ioty = jax.lax.broadcasted_iota(jnp.int32, (blk, padq), 1)
    oha = (ioty == pa).astype(jnp.bfloat16)
    ohb = (ioty == pb).astype(jnp.bfloat16)
    ya = jax.lax.dot_general(
        oha, y_ref[...], (((1,), (0,)), ((), ())),
        preferred_element_type=jnp.float32)
    yb = jax.lax.dot_general(
        ohb, y_ref[...], (((1,), (0,)), ((), ())),
        preferred_element_type=jnp.float32)
    out_ref[...] = xn_ref[...] + shg_ref[...] + s0 * ya + s1 * yb


def _impl(x, w_rms, Ws1, bs1, Ws2, bs2, Wr1, br1, Wr2, br2, centroids,
          interpret):
    b, s, d = x.shape
    ne, _, e = Wr1.shape
    xm = x.reshape(s, d)
    nt = s // TOK_BLK
    p = s * TOPK
    nb = p // BLK + ne
    padq = nb * BLK
    nspec = ((nb + 1 + 15) // 16) * 16

    xn, xnb, aff, scores, eidx = pl.pallas_call(
        _router_body,
        grid=(nt,),
        in_specs=[
            pl.BlockSpec((TOK_BLK, d), lambda t: (t, 0)),
            pl.BlockSpec((1, d), lambda t: (0, 0)),
            pl.BlockSpec((ne, d), lambda t: (0, 0)),
        ],
        out_specs=[
            pl.BlockSpec((TOK_BLK, d), lambda t: (t, 0)),
            pl.BlockSpec((TOK_BLK, d), lambda t: (t, 0)),
            pl.BlockSpec((TOK_BLK, ne), lambda t: (t, 0)),
            pl.BlockSpec((TOK_BLK, TOPK), lambda t: (t, 0)),
            pl.BlockSpec((TOK_BLK, TOPK), lambda t: (t, 0)),
        ],
        out_shape=[
            jax.ShapeDtypeStruct((s, d), jnp.float32),
            jax.ShapeDtypeStruct((s, d), jnp.bfloat16),
            jax.ShapeDtypeStruct((s, ne), jnp.float32),
            jax.ShapeDtypeStruct((s, TOPK), jnp.float32),
            jax.ShapeDtypeStruct((s, TOPK), jnp.int32),
        ],
        interpret=interpret,
    )(xm, w_rms.reshape(1, d), centroids)

    w1b = Wr1.astype(jnp.bfloat16)
    w2b = Wr2.astype(jnp.bfloat16)
    ws1b = Ws1.astype(jnp.bfloat16)
    ws2b = Ws2.astype(jnp.bfloat16)

    shg = pl.pallas_call(
        _shared_body,
        grid=(nt,),
        in_specs=[
            pl.BlockSpec((TOK_BLK, d), lambda t: (t, 0)),
            pl.BlockSpec((1, d, e), lambda t: (0, 0, 0)),
            pl.BlockSpec((1, e, d), lambda t: (0, 0, 0)),
            pl.BlockSpec((1, e), lambda t: (0, 0)),
            pl.BlockSpec((1, d), lambda t: (0, 0)),
        ],
        out_specs=pl.BlockSpec((TOK_BLK, d), lambda t: (t, 0)),
        out_shape=jax.ShapeDtypeStruct((s, d), jnp.float32),
        interpret=interpret,
    )(xnb, ws1b, ws2b, bs1, bs2)

    s1k = _make_s1(s, ne, p, padq, nb, nspec, 0, interpret)
    pos, rowtok, spec = s1k(eidx.reshape(p))

    y = pl.pallas_call(
        _grouped_body,
        grid_spec=pltpu.PrefetchScalarGridSpec(
            num_scalar_prefetch=1,
            grid=(nb,),
            in_specs=[
                pl.BlockSpec((BLK, 1), lambda j, spec: (j, 0)),
                pl.BlockSpec((s, d), lambda j, spec: (0, 0)),
                pl.BlockSpec((1, d, e),
                             lambda j, spec: (_expert_of(j, spec, ne), 0, 0)),
                pl.BlockSpec((1, e, d),
                             lambda j, spec: (_expert_of(j, spec, ne), 0, 0)),
                pl.BlockSpec((ne, e), lambda j, spec: (0, 0)),
                pl.BlockSpec((ne, d), lambda j, spec: (0, 0)),
            ],
            out_specs=pl.BlockSpec((BLK, d), lambda j, spec: (j, 0)),
        ),
        out_shape=jax.ShapeDtypeStruct((padq, d), jnp.bfloat16),
        interpret=interpret,
    )(spec, rowtok.reshape(padq, 1), xnb, w1b, w2b, br1, br2)

    out = pl.pallas_call(
        _combine_body,
        grid=(nt,),
        in_specs=[
            pl.BlockSpec((TOK_BLK, d), lambda t: (t, 0)),
            pl.BlockSpec((TOK_BLK, d), lambda t: (t, 0)),
            pl.BlockSpec((TOK_BLK, TOPK), lambda t: (t, 0)),
            pl.BlockSpec((TOK_BLK, TOPK), lambda t: (t, 0)),
            pl.BlockSpec((padq, d), lambda t: (0, 0)),
        ],
        out_specs=pl.BlockSpec((TOK_BLK, d), lambda t: (t, 0)),
        out_shape=jax.ShapeDtypeStruct((s, d), jnp.float32),
        interpret=interpret,
    )(xn, shg, scores, pos.reshape(s, TOPK), y)

    return out.reshape(b, s, d), aff


def kernel(x, w_rms, Ws1, bs1, Ws2, bs2, Wr1, br1, Wr2, br2, centroids):
    return _impl(x, w_rms, Ws1, bs1, Ws2, bs2, Wr1, br1, Wr2, br2, centroids,
                 interpret=False)


# speedup vs baseline: 1.6670x; 1.0786x over previous
import functools

import jax
import jax.numpy as jnp
from jax import lax
from jax.experimental import pallas as pl
from jax.experimental.pallas import tpu as pltpu
from jax.experimental.pallas import tpu_sc as plsc

EPS = 1e-6
TOK_BLK = 256
TOPK = 2
BLK = 128


def _gelu_exact(y):
    return 0.5 * y * (1.0 + jax.lax.erf(y * 0.7071067811865476))


def _router_body(x_ref, wr_ref, c_ref, xn_ref, xnb_ref, aff_ref, sc_ref,
                 ei_ref):
    xb = x_ref[...]
    ms = jnp.mean(xb * xb, axis=-1, keepdims=True)
    xn = wr_ref[...] * (xb * jax.lax.rsqrt(ms + EPS))
    xn_ref[...] = xn
    xnb_ref[...] = xn.astype(jnp.bfloat16)
    logits = jax.lax.dot_general(
        xn, c_ref[...], (((1,), (1,)), ((), ())),
        preferred_element_type=jnp.float32)
    m = jnp.max(logits, axis=-1, keepdims=True)
    ex = jnp.exp(logits - m)
    aff = ex / jnp.sum(ex, axis=-1, keepdims=True)
    aff_ref[...] = aff
    ne = aff.shape[-1]
    idx = jax.lax.broadcasted_iota(jnp.int32, aff.shape, 1)
    m1 = jnp.max(aff, axis=-1, keepdims=True)
    i1 = jnp.min(jnp.where(aff == m1, idx, ne), axis=-1, keepdims=True)
    oh1 = idx == i1
    a2 = jnp.where(oh1, -jnp.inf, aff)
    m2 = jnp.max(a2, axis=-1, keepdims=True)
    i2 = jnp.min(jnp.where(a2 == m2, idx, ne), axis=-1, keepdims=True)
    sc_ref[...] = jnp.concatenate([m1, m2], axis=1)
    ei_ref[...] = jnp.concatenate([i1, i2], axis=1)


def _shared_body(xnb_ref, w1_ref, w2_ref, bs1_ref, bs2_ref, g_ref):
    x = xnb_ref[...]
    h = jax.lax.dot_general(
        x, w1_ref[0], (((1,), (0,)), ((), ())),
        preferred_element_type=jnp.float32)
    h = h + bs1_ref[...]
    y = jax.lax.dot_general(
        h.astype(jnp.bfloat16), w2_ref[0], (((1,), (0,)), ((), ())),
        preferred_element_type=jnp.float32)
    y = y + bs2_ref[...]
    g_ref[...] = _gelu_exact(y)


def _make_s1(s, ne, p, padq, nb, nspec, tok_off=0, interpret=False):
    n_tiles = 16
    per = p // n_tiles
    nvec = per // 16
    zper = padq // n_tiles
    nsc = per // 128
    mesh = plsc.VectorSubcoreMesh(
        core_axis_name="c", subcore_axis_name="s", num_cores=1)

    @functools.partial(
        pl.kernel,
        out_type=[
            jax.ShapeDtypeStruct((p,), jnp.int32),
            jax.ShapeDtypeStruct((padq,), jnp.int32),
            jax.ShapeDtypeStruct((16,), jnp.int32),
        ],
        mesh=mesh,
        interpret=interpret,
        compiler_params=pltpu.CompilerParams(needs_layout_passes=False),
        scratch_types=[
            pltpu.VMEM((per,), jnp.int32),
            pltpu.VMEM((per,), jnp.int32),
            pltpu.VMEM((nsc, 128), jnp.int32),
            pltpu.VMEM((nsc, 128), jnp.int32),
            pltpu.VMEM((16,), jnp.int32),
            pltpu.VMEM((16,), jnp.int32),
            pltpu.VMEM((zper,), jnp.int32),
            pltpu.VMEM((16, 16), jnp.int32),
            pltpu.VMEM_SHARED((16, 16), jnp.int32),
            pltpu.VMEM_SHARED((padq,), jnp.int32),
        ],
    )
    def s1(ei_hbm, pos_hbm, rowtok_hbm, spec_hbm, e_vm, pos_vm, pos2_vm,
           tok2_vm, row_vm, base_vm, zero_vm, allcnt_vm, counts_sh,
           rowtok_sh):
        wid = lax.axis_index("s")
        base = wid * per
        pltpu.sync_copy(ei_hbm.at[pl.ds(base, per)], e_vm)
        iota = lax.iota(jnp.int32, 16)
        zv = jnp.zeros((16,), jnp.int32)
        ones = jnp.full((16,), 1, jnp.int32)
        c15 = jnp.full((16,), 15, jnp.int32)

        def cvc(x):
            return jnp.full((16,), x, jnp.int32)

        for w in range(n_tiles):
            @pl.when(wid == w)
            def _(w=w):
                row_vm[...] = cvc(w)
        wid_vec = row_vm[...]

        ridx = {k: (iota - cvc(k)) & c15 for k in range(1, 16)}
        geq = {k: iota >= cvc(k) for k in range(1, 16)}

        def rotg(k):
            return plsc.load_gather(row_vm, [ridx[k]])

        def allred(x):
            for k in (1, 2, 4, 8):
                row_vm[...] = x
                x = x + rotg(k)
            return x

        def prefix_incl(x):
            for k in (1, 2, 4, 8):
                row_vm[...] = x
                g = rotg(k)
                x = x + jnp.where(geq[k], g, zv)
            return x

        acc = [zv for _ in range(ne)]
        for i in range(nvec):
            v = e_vm[pl.ds(i * 16, 16)]
            for e in range(ne):
                acc[e] = acc[e] + jnp.where(v == cvc(e), ones, zv)
        hist = zv
        for e in range(ne):
            hist = hist + jnp.where(iota == cvc(e), allred(acc[e]), zv)
        row_vm[...] = hist
        pltpu.sync_copy(row_vm, counts_sh.at[wid])
        for i in range(zper // 16):
            zero_vm[pl.ds(i * 16, 16)] = zv
        plsc.subcore_barrier()
        pltpu.sync_copy(counts_sh, allcnt_vm)
        totals = zv
        prior = zv
        for w in range(n_tiles):
            vw = allcnt_vm[w]
            totals = totals + vw
            prior = prior + vw * jnp.where(cvc(w) < wid_vec, ones, zv)
        nblk = lax.shift_right_logical(totals + cvc(BLK - 1), cvc(7))
        cb_inc = prefix_incl(nblk)
        cb_exc = cb_inc - nblk
        base_vec = cb_exc * cvc(BLK) + prior
        base_vm[...] = cb_inc
        nact_v = plsc.load_gather(base_vm, [cvc(ne - 1)])
        base_vm[...] = base_vec
        for i in range(nvec):
            v = e_vm[pl.ds(i * 16, 16)]
            bv = plsc.load_gather(base_vm, [v])
            row_vm[...] = v
            rank = zv
            cntv = ones
            for k in range(1, 16):
                eq = jnp.where(rotg(k) == v, ones, zv)
                rank = rank + jnp.where(geq[k], eq, zv)
                cntv = cntv + eq
            posv = bv + rank
            plsc.store_scatter(base_vm, [v], bv + cntv)
            pos_vm[pl.ds(i * 16, 16)] = posv
            r, c0 = divmod(i * 16, 128)
            pos2_vm[r, pl.ds(c0, 16)] = posv
            tok2_vm[r, pl.ds(c0, 16)] = lax.shift_right_logical(
                wid_vec * cvc(per) + cvc(i * 16) + iota, ones) + cvc(tok_off)
        pltpu.sync_copy(pos_vm, pos_hbm.at[pl.ds(base, per)])
        pltpu.sync_copy(zero_vm, rowtok_sh.at[pl.ds(wid * zper, zper)])
        plsc.subcore_barrier()
        for ch in range(nsc):
            pltpu.sync_copy(tok2_vm.at[ch], rowtok_sh.at[pos2_vm.at[ch]])
        plsc.subcore_barrier()

        @pl.when(wid == 0)
        def _():
            pltpu.sync_copy(rowtok_sh, rowtok_hbm)
            zero_vm[pl.ds(0, 16)] = cb_inc
            pltpu.sync_copy(zero_vm.at[pl.ds(0, 16)], spec_hbm)

    return s1


def _make_gather(n_rows, d, rows_per, chunk, dtype, interpret=False):
    nchunk = rows_per // chunk
    mesh = plsc.VectorSubcoreMesh(core_axis_name="c", subcore_axis_name="s")

    @functools.partial(
        pl.kernel,
        out_type=jax.ShapeDtypeStruct((n_rows, d), dtype),
        mesh=mesh,
        interpret=interpret,
        compiler_params=pltpu.CompilerParams(needs_layout_passes=False),
        scratch_types=[
            pltpu.VMEM((rows_per,), jnp.int32),
            pltpu.VMEM((chunk, d), dtype),
            pltpu.VMEM((chunk, d), dtype),
            pltpu.SemaphoreType.DMA,
            pltpu.SemaphoreType.DMA,
        ],
    )
    def gk(table_hbm, idx_hbm, out_hbm, idx_vm, buf_a, buf_b, sem_a, sem_b):
        wid = lax.axis_index("s") * 2 + lax.axis_index("c")
        base = wid * rows_per
        pltpu.sync_copy(idx_hbm.at[pl.ds(base, rows_per)], idx_vm)
        bufs = (buf_a, buf_b)
        sems = (sem_a, sem_b)
        cps = []
        for ch in range(nchunk):
            cps.append(pltpu.async_copy(
                table_hbm.at[idx_vm.at[pl.ds(ch * chunk, chunk)]],
                bufs[ch % 2], sems[ch % 2]))
            if ch >= 1:
                cps[ch - 1].wait()
                pltpu.sync_copy(
                    bufs[(ch - 1) % 2],
                    out_hbm.at[pl.ds(base + (ch - 1) * chunk, chunk)])
        cps[nchunk - 1].wait()
        pltpu.sync_copy(
            bufs[(nchunk - 1) % 2],
            out_hbm.at[pl.ds(base + (nchunk - 1) * chunk, chunk)])

    return gk


def _expert_of(j, spec_ref, ne):
    e = jnp.int32(0)
    for k in range(ne - 1):
        e = e + jnp.where(j >= spec_ref[k], 1, 0).astype(jnp.int32)
    return e


def _grouped_body(spec_ref, rt_ref, xnb_ref, w1_ref, w2_ref, br1_ref,
                  br2_ref, y_ref):
    j = pl.program_id(0)
    ne = br1_ref.shape[0]
    nact = spec_ref[ne - 1]
    be = _expert_of(j, spec_ref, ne)

    @pl.when(j < nact)
    def _():
        rt = rt_ref[...]
        s = xnb_ref.shape[0]
        ioty = jax.lax.broadcasted_iota(jnp.int32, (rt.shape[0], s), 1)
        oh = (ioty == rt).astype(jnp.bfloat16)
        xg = jax.lax.dot_general(
            oh, xnb_ref[...], (((1,), (0,)), ((), ())),
            preferred_element_type=jnp.float32).astype(jnp.bfloat16)
        h = jax.lax.dot_general(
            xg, w1_ref[0], (((1,), (0,)), ((), ())),
            preferred_element_type=jnp.float32)
        h = h + br1_ref[pl.ds(be, 1), :]
        y = jax.lax.dot_general(
            h.astype(jnp.bfloat16), w2_ref[0], (((1,), (0,)), ((), ())),
            preferred_element_type=jnp.float32)
        y = y + br2_ref[pl.ds(be, 1), :]
        y_ref[...] = _gelu_exact(y).astype(jnp.bfloat16)


def _combine_body(xn_ref, shg_ref, sc_ref, pos_ref, y_ref, out_ref):
    padq = y_ref.shape[0]
    blk = xn_ref.shape[0]
    s0 = sc_ref[:, 0:1]
    s1 = sc_ref[:, 1:2]
    pa = pos_ref[:, 0:1]
    pb = pos_ref[:, 1:2]
    ioty = jax.lax.broadcasted_iota(jnp.int32, (blk, padq), 1)
    ohw = (jnp.where(ioty == pa, s0, 0.0)
           + jnp.where(ioty == pb, s1, 0.0)).astype(jnp.bfloat16)
    routed = jax.lax.dot_general(
        ohw, y_ref[...], (((1,), (0,)), ((), ())),
        preferred_element_type=jnp.float32)
    out_ref[...] = xn_ref[...] + shg_ref[...] + routed


def _impl(x, w_rms, Ws1, bs1, Ws2, bs2, Wr1, br1, Wr2, br2, centroids,
          interpret):
    b, s, d = x.shape
    ne, _, e = Wr1.shape
    xm = x.reshape(s, d)
    nt = s // TOK_BLK
    p = s * TOPK
    nb = p // BLK + ne
    padq = nb * BLK
    nspec = ((nb + 1 + 15) // 16) * 16

    xn, xnb, aff, scores, eidx = pl.pallas_call(
        _router_body,
        grid=(nt,),
        in_specs=[
            pl.BlockSpec((TOK_BLK, d), lambda t: (t, 0)),
            pl.BlockSpec((1, d), lambda t: (0, 0)),
            pl.BlockSpec((ne, d), lambda t: (0, 0)),
        ],
        out_specs=[
            pl.BlockSpec((TOK_BLK, d), lambda t: (t, 0)),
            pl.BlockSpec((TOK_BLK, d), lambda t: (t, 0)),
            pl.BlockSpec((TOK_BLK, ne), lambda t: (t, 0)),
            pl.BlockSpec((TOK_BLK, TOPK), lambda t: (t, 0)),
            pl.BlockSpec((TOK_BLK, TOPK), lambda t: (t, 0)),
        ],
        out_shape=[
            jax.ShapeDtypeStruct((s, d), jnp.float32),
            jax.ShapeDtypeStruct((s, d), jnp.bfloat16),
            jax.ShapeDtypeStruct((s, ne), jnp.float32),
            jax.ShapeDtypeStruct((s, TOPK), jnp.float32),
            jax.ShapeDtypeStruct((s, TOPK), jnp.int32),
        ],
        interpret=interpret,
    )(xm, w_rms.reshape(1, d), centroids)

    w1b = Wr1.astype(jnp.bfloat16)
    w2b = Wr2.astype(jnp.bfloat16)
    ws1b = Ws1.astype(jnp.bfloat16)
    ws2b = Ws2.astype(jnp.bfloat16)

    shg = pl.pallas_call(
        _shared_body,
        grid=(nt,),
        in_specs=[
            pl.BlockSpec((TOK_BLK, d), lambda t: (t, 0)),
            pl.BlockSpec((1, d, e), lambda t: (0, 0, 0)),
            pl.BlockSpec((1, e, d), lambda t: (0, 0, 0)),
            pl.BlockSpec((1, e), lambda t: (0, 0)),
            pl.BlockSpec((1, d), lambda t: (0, 0)),
        ],
        out_specs=pl.BlockSpec((TOK_BLK, d), lambda t: (t, 0)),
        out_shape=jax.ShapeDtypeStruct((s, d), jnp.float32),
        interpret=interpret,
    )(xnb, ws1b, ws2b, bs1, bs2)

    s1k = _make_s1(s, ne, p, padq, nb, nspec, 0, interpret)
    pos, rowtok, spec = s1k(eidx.reshape(p))

    y = pl.pallas_call(
        _grouped_body,
        grid_spec=pltpu.PrefetchScalarGridSpec(
            num_scalar_prefetch=1,
            grid=(nb,),
            in_specs=[
                pl.BlockSpec((BLK, 1), lambda j, spec: (j, 0)),
                pl.BlockSpec((s, d), lambda j, spec: (0, 0)),
                pl.BlockSpec((1, d, e),
                             lambda j, spec: (_expert_of(j, spec, ne), 0, 0)),
                pl.BlockSpec((1, e, d),
                             lambda j, spec: (_expert_of(j, spec, ne), 0, 0)),
                pl.BlockSpec((ne, e), lambda j, spec: (0, 0)),
                pl.BlockSpec((ne, d), lambda j, spec: (0, 0)),
            ],
            out_specs=pl.BlockSpec((BLK, d), lambda j, spec: (j, 0)),
        ),
        out_shape=jax.ShapeDtypeStruct((padq, d), jnp.bfloat16),
        interpret=interpret,
    )(spec, rowtok.reshape(padq, 1), xnb, w1b, w2b, br1, br2)

    out = pl.pallas_call(
        _combine_body,
        grid=(nt,),
        in_specs=[
            pl.BlockSpec((TOK_BLK, d), lambda t: (t, 0)),
            pl.BlockSpec((TOK_BLK, d), lambda t: (t, 0)),
            pl.BlockSpec((TOK_BLK, TOPK), lambda t: (t, 0)),
            pl.BlockSpec((TOK_BLK, TOPK), lambda t: (t, 0)),
            pl.BlockSpec((padq, d), lambda t: (0, 0)),
        ],
        out_specs=pl.BlockSpec((TOK_BLK, d), lambda t: (t, 0)),
        out_shape=jax.ShapeDtypeStruct((s, d), jnp.float32),
        interpret=interpret,
    )(xn, shg, scores, pos.reshape(s, TOPK), y)

    return out.reshape(b, s, d), aff


def kernel(x, w_rms, Ws1, bs1, Ws2, bs2, Wr1, br1, Wr2, br2, centroids):
    return _impl(x, w_rms, Ws1, bs1, Ws2, bs2, Wr1, br1, Wr2, br2, centroids,
                 interpret=False)


# speedup vs baseline: 1.7051x; 1.0229x over previous
import functools

import jax
import jax.numpy as jnp
from jax import lax
from jax.experimental import pallas as pl
from jax.experimental.pallas import tpu as pltpu
from jax.experimental.pallas import tpu_sc as plsc

EPS = 1e-6
TOK_BLK = 256
TOPK = 2
BLK = 128


def _gelu_exact(y):
    return 0.5 * y * (1.0 + jax.lax.erf(y * 0.7071067811865476))


def _router_body(x_ref, wr_ref, c_ref, xn_ref, xnb_ref, aff_ref, sc_ref,
                 ei_ref):
    xb = x_ref[...]
    ms = jnp.mean(xb * xb, axis=-1, keepdims=True)
    xn = wr_ref[...] * (xb * jax.lax.rsqrt(ms + EPS))
    xn_ref[...] = xn
    xnb_ref[...] = xn.astype(jnp.bfloat16)
    logits = jax.lax.dot_general(
        xn, c_ref[...], (((1,), (1,)), ((), ())),
        preferred_element_type=jnp.float32)
    m = jnp.max(logits, axis=-1, keepdims=True)
    ex = jnp.exp(logits - m)
    aff = ex / jnp.sum(ex, axis=-1, keepdims=True)
    aff_ref[...] = aff
    ne = aff.shape[-1]
    idx = jax.lax.broadcasted_iota(jnp.int32, aff.shape, 1)
    m1 = jnp.max(aff, axis=-1, keepdims=True)
    i1 = jnp.min(jnp.where(aff == m1, idx, ne), axis=-1, keepdims=True)
    oh1 = idx == i1
    a2 = jnp.where(oh1, -jnp.inf, aff)
    m2 = jnp.max(a2, axis=-1, keepdims=True)
    i2 = jnp.min(jnp.where(a2 == m2, idx, ne), axis=-1, keepdims=True)
    sc_ref[...] = jnp.concatenate([m1, m2], axis=1)
    ei_ref[...] = jnp.concatenate([i1, i2], axis=1)


def _make_s1(s, ne, p, padq, nb, nspec, tok_off=0, interpret=False):
    n_tiles = 16
    per = p // n_tiles
    nvec = per // 16
    zper = padq // n_tiles
    nsc = per // 128
    mesh = plsc.VectorSubcoreMesh(
        core_axis_name="c", subcore_axis_name="s", num_cores=1)

    @functools.partial(
        pl.kernel,
        out_type=[
            jax.ShapeDtypeStruct((p,), jnp.int32),
            jax.ShapeDtypeStruct((padq,), jnp.int32),
            jax.ShapeDtypeStruct((16,), jnp.int32),
        ],
        mesh=mesh,
        interpret=interpret,
        compiler_params=pltpu.CompilerParams(needs_layout_passes=False),
        scratch_types=[
            pltpu.VMEM((per,), jnp.int32),
            pltpu.VMEM((per,), jnp.int32),
            pltpu.VMEM((nsc, 128), jnp.int32),
            pltpu.VMEM((nsc, 128), jnp.int32),
            pltpu.VMEM((16,), jnp.int32),
            pltpu.VMEM((16,), jnp.int32),
            pltpu.VMEM((zper,), jnp.int32),
            pltpu.VMEM((16, 16), jnp.int32),
            pltpu.VMEM_SHARED((16, 16), jnp.int32),
            pltpu.VMEM_SHARED((padq,), jnp.int32),
        ],
    )
    def s1(ei_hbm, pos_hbm, rowtok_hbm, spec_hbm, e_vm, pos_vm, pos2_vm,
           tok2_vm, row_vm, base_vm, zero_vm, allcnt_vm, counts_sh,
           rowtok_sh):
        wid = lax.axis_index("s")
        base = wid * per
        pltpu.sync_copy(ei_hbm.at[pl.ds(base, per)], e_vm)
        iota = lax.iota(jnp.int32, 16)
        zv = jnp.zeros((16,), jnp.int32)
        ones = jnp.full((16,), 1, jnp.int32)
        c15 = jnp.full((16,), 15, jnp.int32)

        def cvc(x):
            return jnp.full((16,), x, jnp.int32)

        for w in range(n_tiles):
            @pl.when(wid == w)
            def _(w=w):
                row_vm[...] = cvc(w)
        wid_vec = row_vm[...]

        ridx = {k: (iota - cvc(k)) & c15 for k in range(1, 16)}
        geq = {k: iota >= cvc(k) for k in range(1, 16)}

        def rotg(k):
            return plsc.load_gather(row_vm, [ridx[k]])

        def allred(x):
            for k in (1, 2, 4, 8):
                row_vm[...] = x
                x = x + rotg(k)
            return x

        def prefix_incl(x):
            for k in (1, 2, 4, 8):
                row_vm[...] = x
                g = rotg(k)
                x = x + jnp.where(geq[k], g, zv)
            return x

        acc = [zv for _ in range(ne)]
        for i in range(nvec):
            v = e_vm[pl.ds(i * 16, 16)]
            for e in range(ne):
                acc[e] = acc[e] + jnp.where(v == cvc(e), ones, zv)
        hist = zv
        for e in range(ne):
            hist = hist + jnp.where(iota == cvc(e), allred(acc[e]), zv)
        row_vm[...] = hist
        pltpu.sync_copy(row_vm, counts_sh.at[wid])
        for i in range(zper // 16):
            zero_vm[pl.ds(i * 16, 16)] = zv
        plsc.subcore_barrier()
        pltpu.sync_copy(counts_sh, allcnt_vm)
        totals = zv
        prior = zv
        for w in range(n_tiles):
            vw = allcnt_vm[w]
            totals = totals + vw
            prior = prior + vw * jnp.where(cvc(w) < wid_vec, ones, zv)
        nblk = lax.shift_right_logical(totals + cvc(BLK - 1), cvc(7))
        cb_inc = prefix_incl(nblk)
        cb_exc = cb_inc - nblk
        base_vec = cb_exc * cvc(BLK) + prior
        base_vm[...] = cb_inc
        nact_v = plsc.load_gather(base_vm, [cvc(ne - 1)])
        base_vm[...] = base_vec
        for i in range(nvec):
            v = e_vm[pl.ds(i * 16, 16)]
            bv = plsc.load_gather(base_vm, [v])
            row_vm[...] = v
            rank = zv
            cntv = ones
            for k in range(1, 16):
                eq = jnp.where(rotg(k) == v, ones, zv)
                rank = rank + jnp.where(geq[k], eq, zv)
                cntv = cntv + eq
            posv = bv + rank
            plsc.store_scatter(base_vm, [v], bv + cntv)
            pos_vm[pl.ds(i * 16, 16)] = posv
            r, c0 = divmod(i * 16, 128)
            pos2_vm[r, pl.ds(c0, 16)] = posv
            tok2_vm[r, pl.ds(c0, 16)] = lax.shift_right_logical(
                wid_vec * cvc(per) + cvc(i * 16) + iota, ones) + cvc(tok_off)
        pltpu.sync_copy(pos_vm, pos_hbm.at[pl.ds(base, per)])
        pltpu.sync_copy(zero_vm, rowtok_sh.at[pl.ds(wid * zper, zper)])
        plsc.subcore_barrier()
        for ch in range(nsc):
            pltpu.sync_copy(tok2_vm.at[ch], rowtok_sh.at[pos2_vm.at[ch]])
        plsc.subcore_barrier()

        @pl.when(wid == 0)
        def _():
            pltpu.sync_copy(rowtok_sh, rowtok_hbm)
            zero_vm[pl.ds(0, 16)] = cb_inc
            pltpu.sync_copy(zero_vm.at[pl.ds(0, 16)], spec_hbm)

    return s1


def _expert_of(j, spec_ref, ne):
    e = jnp.int32(0)
    for k in range(ne - 1):
        e = e + jnp.where(j >= spec_ref[k], 1, 0).astype(jnp.int32)
    return e


def _grouped_body(spec_ref, rt_ref, xnb_ref, w1_ref, w2_ref, br1_ref,
                  br2_ref, y_ref):
    j = pl.program_id(0)
    ne = br1_ref.shape[0]
    nact = spec_ref[ne - 1]
    be = _expert_of(j, spec_ref, ne)

    @pl.when(j < nact)
    def _():
        rt = rt_ref[...]
        s = xnb_ref.shape[0]
        ioty = jax.lax.broadcasted_iota(jnp.int32, (rt.shape[0], s), 1)
        oh = (ioty == rt).astype(jnp.bfloat16)
        xg = jax.lax.dot_general(
            oh, xnb_ref[...], (((1,), (0,)), ((), ())),
            preferred_element_type=jnp.float32).astype(jnp.bfloat16)
        h = jax.lax.dot_general(
            xg, w1_ref[0], (((1,), (0,)), ((), ())),
            preferred_element_type=jnp.float32)
        h = h + br1_ref[pl.ds(be, 1), :]
        y = jax.lax.dot_general(
            h.astype(jnp.bfloat16), w2_ref[0], (((1,), (0,)), ((), ())),
            preferred_element_type=jnp.float32)
        y = y + br2_ref[pl.ds(be, 1), :]
        y_ref[...] = _gelu_exact(y).astype(jnp.bfloat16)


def _combine_body(xn_ref, xnb_ref, sc_ref, pos_ref, y_ref, w1_ref,
                  w2_ref, bs1_ref, bs2_ref, out_ref):
    padq = y_ref.shape[0]
    blk = xn_ref.shape[0]
    x = xnb_ref[...]
    h = jax.lax.dot_general(
        x, w1_ref[0], (((1,), (0,)), ((), ())),
        preferred_element_type=jnp.float32)
    h = h + bs1_ref[...]
    ys = jax.lax.dot_general(
        h.astype(jnp.bfloat16), w2_ref[0], (((1,), (0,)), ((), ())),
        preferred_element_type=jnp.float32)
    shg = _gelu_exact(ys + bs2_ref[...])
    s0 = sc_ref[:, 0:1]
    s1 = sc_ref[:, 1:2]
    pa = pos_ref[:, 0:1]
    pb = pos_ref[:, 1:2]
    ioty = jax.lax.broadcasted_iota(jnp.int32, (blk, padq), 1)
    ohw = (jnp.where(ioty == pa, s0, 0.0)
           + jnp.where(ioty == pb, s1, 0.0)).astype(jnp.bfloat16)
    routed = jax.lax.dot_general(
        ohw, y_ref[...], (((1,), (0,)), ((), ())),
        preferred_element_type=jnp.float32)
    out_ref[...] = xn_ref[...] + shg + routed


def _impl(x, w_rms, Ws1, bs1, Ws2, bs2, Wr1, br1, Wr2, br2, centroids,
          interpret):
    b, s, d = x.shape
    ne, _, e = Wr1.shape
    xm = x.reshape(s, d)
    nt = s // TOK_BLK
    p = s * TOPK
    nb = p // BLK + ne
    padq = nb * BLK
    nspec = ((nb + 1 + 15) // 16) * 16

    xn, xnb, aff, scores, eidx = pl.pallas_call(
        _router_body,
        grid=(nt,),
        in_specs=[
            pl.BlockSpec((TOK_BLK, d), lambda t: (t, 0)),
            pl.BlockSpec((1, d), lambda t: (0, 0)),
            pl.BlockSpec((ne, d), lambda t: (0, 0)),
        ],
        out_specs=[
            pl.BlockSpec((TOK_BLK, d), lambda t: (t, 0)),
            pl.BlockSpec((TOK_BLK, d), lambda t: (t, 0)),
            pl.BlockSpec((TOK_BLK, ne), lambda t: (t, 0)),
            pl.BlockSpec((TOK_BLK, TOPK), lambda t: (t, 0)),
            pl.BlockSpec((TOK_BLK, TOPK), lambda t: (t, 0)),
        ],
        out_shape=[
            jax.ShapeDtypeStruct((s, d), jnp.float32),
            jax.ShapeDtypeStruct((s, d), jnp.bfloat16),
            jax.ShapeDtypeStruct((s, ne), jnp.float32),
            jax.ShapeDtypeStruct((s, TOPK), jnp.float32),
            jax.ShapeDtypeStruct((s, TOPK), jnp.int32),
        ],
        interpret=interpret,
    )(xm, w_rms.reshape(1, d), centroids)

    w1b = Wr1.astype(jnp.bfloat16)
    w2b = Wr2.astype(jnp.bfloat16)
    ws1b = Ws1.astype(jnp.bfloat16)
    ws2b = Ws2.astype(jnp.bfloat16)

    s1k = _make_s1(s, ne, p, padq, nb, nspec, 0, interpret)
    pos, rowtok, spec = s1k(eidx.reshape(p))

    y = pl.pallas_call(
        _grouped_body,
        grid_spec=pltpu.PrefetchScalarGridSpec(
            num_scalar_prefetch=1,
            grid=(nb,),
            in_specs=[
                pl.BlockSpec((BLK, 1), lambda j, spec: (j, 0)),
                pl.BlockSpec((s, d), lambda j, spec: (0, 0)),
                pl.BlockSpec((1, d, e),
                             lambda j, spec: (_expert_of(j, spec, ne), 0, 0)),
                pl.BlockSpec((1, e, d),
                             lambda j, spec: (_expert_of(j, spec, ne), 0, 0)),
                pl.BlockSpec((ne, e), lambda j, spec: (0, 0)),
                pl.BlockSpec((ne, d), lambda j, spec: (0, 0)),
            ],
            out_specs=pl.BlockSpec((BLK, d), lambda j, spec: (j, 0)),
        ),
        out_shape=jax.ShapeDtypeStruct((padq, d), jnp.bfloat16),
        interpret=interpret,
    )(spec, rowtok.reshape(padq, 1), xnb, w1b, w2b, br1, br2)

    out = pl.pallas_call(
        _combine_body,
        grid=(nt,),
        in_specs=[
            pl.BlockSpec((TOK_BLK, d), lambda t: (t, 0)),
            pl.BlockSpec((TOK_BLK, d), lambda t: (t, 0)),
            pl.BlockSpec((TOK_BLK, TOPK), lambda t: (t, 0)),
            pl.BlockSpec((TOK_BLK, TOPK), lambda t: (t, 0)),
            pl.BlockSpec((padq, d), lambda t: (0, 0)),
            pl.BlockSpec((1, d, e), lambda t: (0, 0, 0)),
            pl.BlockSpec((1, e, d), lambda t: (0, 0, 0)),
            pl.BlockSpec((1, e), lambda t: (0, 0)),
            pl.BlockSpec((1, d), lambda t: (0, 0)),
        ],
        out_specs=pl.BlockSpec((TOK_BLK, d), lambda t: (t, 0)),
        out_shape=jax.ShapeDtypeStruct((s, d), jnp.float32),
        interpret=interpret,
    )(xn, xnb, scores, pos.reshape(s, TOPK), y, ws1b, ws2b, bs1, bs2)

    return out.reshape(b, s, d), aff


def kernel(x, w_rms, Ws1, bs1, Ws2, bs2, Wr1, br1, Wr2, br2, centroids):
    return _impl(x, w_rms, Ws1, bs1, Ws2, bs2, Wr1, br1, Wr2, br2, centroids,
                 interpret=False)
